# baseline (speedup 1.0000x reference)
import functools

import jax
import jax.numpy as jnp
from jax import lax
from jax.experimental import pallas as pl
from jax.experimental.pallas import tpu as pltpu
from jax.experimental.pallas import tpu_sc as plsc

_B, _S, _D, _T = 4, 2048, 1024, 128
_N = _B * _S
_N_SC = 2048
_N_TC = _N - _N_SC
_NC, _NS = 2, 16
_NW = _NC * _NS
_NCG = 8
_CW = _D // _NCG
_NTG = _NW // _NCG
_TPG = _N_SC // _NTG
_CH = 256
_NCH = _TPG // _CH
_RPW = _N_SC // _NW
_TCB = 2048
_TCG = _N_TC // _TCB


def _sc_segsum(feat2d, lab1d):
    mesh = plsc.VectorSubcoreMesh(core_axis_name="c", subcore_axis_name="s")

    @functools.partial(
        pl.kernel,
        mesh=mesh,
        out_type=(
            jax.ShapeDtypeStruct((_NTG, _T, _D), jnp.float32),
            jax.ShapeDtypeStruct((_NW, _T, 16), jnp.float32),
        ),
        scratch_types=[
            pltpu.VMEM((_TPG,), jnp.int32),
            pltpu.VMEM((2, _CH, _CW), jnp.float32),
            pltpu.VMEM((_T, _CW), jnp.float32),
            pltpu.VMEM((_T, 16), jnp.float32),
            pltpu.SemaphoreType.DMA,
            pltpu.SemaphoreType.DMA,
        ],
    )
    def k(feat_hbm, lab_hbm, out_sum, out_cnt,
          lab_v, buf_v, acc_v, cnt_v, sem0, sem1):
        c = lax.axis_index("c")
        s = lax.axis_index("s")
        w = s * _NC + c
        tg = w // _NCG
        cg = w % _NCG
        col0 = cg * _CW
        row0 = tg * _TPG

        lab_cp = pltpu.async_copy(
            lab_hbm.at[row0 // _S, pl.ds(row0 % _S, _TPG)], lab_v, sem0)
        pltpu.async_copy(feat_hbm.at[pl.ds(row0, _CH), pl.ds(col0, _CW)],
                         buf_v.at[0], sem1)

        z16 = jnp.zeros((16,), jnp.float32)

        @plsc.parallel_loop(0, _T)
        def zbody(r):
            for cc in range(_CW // 16):
                acc_v[r, pl.ds(cc * 16, 16)] = z16
            cnt_v[r] = z16

        lab_cp.wait()

        one16 = jnp.ones((16,), jnp.float32)

        @plsc.parallel_loop(0, _RPW // 16)
        def cgrp(g):
            lvec = lab_v[pl.ds(cg * _RPW + g * 16, 16)]
            for j in range(16):
                plsc.addupdate(cnt_v.at[lvec[j]], one16)

        def chunk(ch, carry):
            pltpu.make_async_copy(
                feat_hbm.at[pl.ds(row0 + ch * _CH, _CH), pl.ds(col0, _CW)],
                buf_v.at[ch % 2], sem1).wait()

            @pl.when(ch + 1 < _NCH)
            def _():
                pltpu.async_copy(
                    feat_hbm.at[pl.ds(row0 + (ch + 1) * _CH, _CH),
                                pl.ds(col0, _CW)],
                    buf_v.at[(ch + 1) % 2], sem1)

            @plsc.parallel_loop(0, _CH // 16, unroll=4)
            def tokgrp(g):
                lvec = lab_v[pl.ds(ch * _CH + g * 16, 16)]
                for j in range(16):
                    lab = lvec[j]
                    vs = [buf_v[ch % 2, g * 16 + j, pl.ds(cc * 16, 16)]
                          for cc in range(_CW // 16)]
                    for cc in range(_CW // 16):
                        plsc.addupdate(acc_v.at[lab, pl.ds(cc * 16, 16)],
                                       vs[cc])

            return carry

        lax.fori_loop(0, _NCH, chunk, 0)

        pltpu.sync_copy(acc_v, out_sum.at[tg, :, pl.ds(col0, _CW)])
        pltpu.sync_copy(cnt_v, out_cnt.at[w])

    return k(feat2d, lab1d)


def _proto_rank(proto_ref, pp_ref, disc_ref):
    f32 = jnp.float32
    proto = proto_ref[...]
    pp = lax.dot_general(proto, proto, (((1,), (1,)), ((), ())),
                         preferred_element_type=f32)
    pp_ref[...] = pp

    ri = lax.broadcasted_iota(jnp.int32, (_T, _T), 0)
    ci = lax.broadcasted_iota(jnp.int32, (_T, _T), 1)
    eye = (ri == ci).astype(f32)
    pn2_col = jnp.sum(pp * eye, axis=1, keepdims=True)
    pn2_row = jnp.sum(pp * eye, axis=0, keepdims=True)
    pn_col = jnp.maximum(jnp.sqrt(pn2_col), 1e-8)
    pn_row = jnp.maximum(jnp.sqrt(pn2_row), 1e-8)
    sim = pp / (pn_col * pn_row)

    a3 = sim[:, :, None]
    b3 = sim[:, None, :]
    ki = lax.broadcasted_iota(jnp.int32, (_T, _T, _T), 1)
    ji = lax.broadcasted_iota(jnp.int32, (_T, _T, _T), 2)
    gt = (a3 > b3).astype(f32)
    eq = jnp.logical_and(a3 == b3, ki < ji).astype(f32)
    rank = jnp.sum(gt + eq, axis=1)
    disc_ref[...] = jnp.log(rank + 2.0) * 1.4426950408889634


def _tc1_body(lab_ref, x_ref, proto_ref, fsum_ref, cnt_ref, pp_ref, disc_ref):
    f32 = jnp.float32
    i = pl.program_id(0)

    @pl.when(i == 0)
    def _proto_side():
        _proto_rank(proto_ref, pp_ref, disc_ref)

    lab_row = lab_ref[0]
    lab3 = jnp.broadcast_to(lab_row, (_T, _TCB))
    ti = lax.broadcasted_iota(jnp.int32, (_T, _TCB), 0)
    oh_bool = lab3 == ti
    oht = oh_bool.astype(f32)

    part = lax.dot_general(oht, x_ref[...], (((1,), (0,)), ((), ())),
                           preferred_element_type=f32)
    pcnt = jnp.sum(oh_bool.astype(f32), axis=1, keepdims=True)

    @pl.when(i == 0)
    def _init():
        fsum_ref[...] = part
        cnt_ref[...] = pcnt

    @pl.when(i > 0)
    def _acc():
        fsum_ref[...] = fsum_ref[...] + part
        cnt_ref[...] = cnt_ref[...] + pcnt


def _tc2_body(psum_ref, pcnt_ref, fsum_ref, cntc_ref, w_ref, b_ref,
              proto_ref, pp_ref, disc_ref, temp_ref, out_ref):
    f32 = jnp.float32
    featsum = jnp.sum(psum_ref[...], axis=0) + fsum_ref[...]
    counts = jnp.sum(pcnt_ref[...], axis=0)[:, 0:1] + cntc_ref[...]
    sums = lax.dot_general(featsum, w_ref[...], (((1,), (1,)), ((), ())),
                           preferred_element_type=f32)
    sums = sums + counts * b_ref[...]
    means = sums / jnp.maximum(counts, 1.0)

    mp = lax.dot_general(means, proto_ref[...], (((1,), (1,)), ((), ())),
                         preferred_element_type=f32)
    pp = pp_ref[...]

    ri = lax.broadcasted_iota(jnp.int32, (_T, _T), 0)
    ci = lax.broadcasted_iota(jnp.int32, (_T, _T), 1)
    eye = (ri == ci).astype(f32)

    m2 = jnp.sum(means * means, axis=1, keepdims=True)
    nm = jnp.maximum(jnp.sqrt(m2), 1e-8)
    pn2_row = jnp.sum(pp * eye, axis=0, keepdims=True)
    pn_row = jnp.maximum(jnp.sqrt(pn2_row), 1e-8)

    cos_mp = mp / (nm * pn_row)
    temp = temp_ref[0, 0]
    apd = (-(1.0 - cos_mp) / temp) / disc_ref[...]

    rmax = jnp.max(apd, axis=1, keepdims=True)
    lse = jnp.log(jnp.sum(jnp.exp(apd - rmax), axis=1, keepdims=True)) + rmax
    diag_ap = jnp.sum(apd * eye, axis=1, keepdims=True)
    loss_i = lse - diag_ap
    present = (counts > 0).astype(f32)
    total = jnp.sum(loss_i * present, axis=0, keepdims=True)
    out_ref[...] = total / _T


def kernel(features, labels, W, b, proto, temperature=0.3):
    feat2d = features.reshape(_N, _D)
    lab2d = labels.astype(jnp.int32)

    psum, pcnt = _sc_segsum(feat2d, lab2d)

    fsum_tc, cnt_tc, pp, disc = pl.pallas_call(
        _tc1_body,
        grid=(_TCG,),
        out_shape=(
            jax.ShapeDtypeStruct((_T, _D), jnp.float32),
            jax.ShapeDtypeStruct((_T, 1), jnp.float32),
            jax.ShapeDtypeStruct((_T, _T), jnp.float32),
            jax.ShapeDtypeStruct((_T, _T), jnp.float32),
        ),
        in_specs=[
            pl.BlockSpec((1, 1, _S), lambda i: (i + _N_SC // _S, 0, 0)),
            pl.BlockSpec((_TCB, _D), lambda i: (i + _N_SC // _TCB, 0)),
            pl.BlockSpec((_T, _T), lambda i: (0, 0)),
        ],
        out_specs=(
            pl.BlockSpec((_T, _D), lambda i: (0, 0)),
            pl.BlockSpec((_T, 1), lambda i: (0, 0)),
            pl.BlockSpec((_T, _T), lambda i: (0, 0)),
            pl.BlockSpec((_T, _T), lambda i: (0, 0)),
        ),
    )(lab2d.reshape(_B, 1, _S), feat2d, proto)

    b_row = b.reshape(1, _T).astype(jnp.float32)
    t11 = jnp.asarray(temperature, jnp.float32).reshape(1, 1)
    out = pl.pallas_call(
        _tc2_body,
        out_shape=jax.ShapeDtypeStruct((1, 1), jnp.float32),
        in_specs=[pl.BlockSpec(memory_space=pltpu.VMEM)] * 9
        + [pl.BlockSpec(memory_space=pltpu.SMEM)],
        out_specs=pl.BlockSpec(memory_space=pltpu.VMEM),
    )(psum, pcnt, fsum_tc, cnt_tc, W.astype(jnp.float32), b_row,
      proto, pp, disc, t11)
    return out.reshape(1)


# speedup vs baseline: 1.0777x; 1.0777x over previous
import functools

import jax
import jax.numpy as jnp
from jax import lax
from jax.experimental import pallas as pl
from jax.experimental.pallas import tpu as pltpu
from jax.experimental.pallas import tpu_sc as plsc

_B, _S, _D, _T = 4, 2048, 1024, 128
_N = _B * _S
_N_SC = 2048
_N_TC = _N - _N_SC
_NC, _NS = 2, 16
_NW = _NC * _NS
_NCG = 8
_CW = _D // _NCG
_NTG = _NW // _NCG
_TPG = _N_SC // _NTG
_CH = 256
_NCH = _TPG // _CH
_RPW = _N_SC // _NW
_TCB = 2048
_TCG = _N_TC // _TCB


def _sc_segsum(feat2d, lab1d):
    mesh = plsc.VectorSubcoreMesh(core_axis_name="c", subcore_axis_name="s")

    @functools.partial(
        pl.kernel,
        mesh=mesh,
        out_type=jax.ShapeDtypeStruct((_NTG, _T, _D), jnp.float32),
        scratch_types=[
            pltpu.VMEM((_TPG,), jnp.int32),
            pltpu.VMEM((2, _CH, _CW), jnp.float32),
            pltpu.VMEM((_T, _CW), jnp.float32),
            pltpu.SemaphoreType.DMA,
            pltpu.SemaphoreType.DMA,
        ],
    )
    def k(feat_hbm, lab_hbm, out_sum,
          lab_v, buf_v, acc_v, sem0, sem1):
        c = lax.axis_index("c")
        s = lax.axis_index("s")
        w = s * _NC + c
        tg = w // _NCG
        cg = w % _NCG
        col0 = cg * _CW
        row0 = tg * _TPG

        lab_cp = pltpu.async_copy(
            lab_hbm.at[row0 // _S, pl.ds(row0 % _S, _TPG)], lab_v, sem0)
        pltpu.async_copy(feat_hbm.at[pl.ds(row0, _CH), pl.ds(col0, _CW)],
                         buf_v.at[0], sem1)

        z16 = jnp.zeros((16,), jnp.float32)

        @plsc.parallel_loop(0, _T)
        def zbody(r):
            for cc in range(_CW // 16):
                acc_v[r, pl.ds(cc * 16, 16)] = z16

        lab_cp.wait()

        def chunk(ch, carry):
            pltpu.make_async_copy(
                feat_hbm.at[pl.ds(row0 + ch * _CH, _CH), pl.ds(col0, _CW)],
                buf_v.at[ch % 2], sem1).wait()

            @pl.when(ch + 1 < _NCH)
            def _():
                pltpu.async_copy(
                    feat_hbm.at[pl.ds(row0 + (ch + 1) * _CH, _CH),
                                pl.ds(col0, _CW)],
                    buf_v.at[(ch + 1) % 2], sem1)

            @plsc.parallel_loop(0, _CH // 16, unroll=2)
            def tokgrp(g):
                lvec = lab_v[pl.ds(ch * _CH + g * 16, 16)]
                for j in range(16):
                    lab = lvec[j]
                    vs = [buf_v[ch % 2, g * 16 + j, pl.ds(cc * 16, 16)]
                          for cc in range(_CW // 16)]
                    for cc in range(_CW // 16):
                        plsc.addupdate(acc_v.at[lab, pl.ds(cc * 16, 16)],
                                       vs[cc])

            return carry

        lax.fori_loop(0, _NCH, chunk, 0)

        pltpu.sync_copy(acc_v, out_sum.at[tg, :, pl.ds(col0, _CW)])

    return k(feat2d, lab1d)


def _proto_rank(proto_ref, pp_ref, disc_ref):
    f32 = jnp.float32
    proto = proto_ref[...]
    pp = lax.dot_general(proto, proto, (((1,), (1,)), ((), ())),
                         preferred_element_type=f32)
    pp_ref[...] = pp

    ri = lax.broadcasted_iota(jnp.int32, (_T, _T), 0)
    ci = lax.broadcasted_iota(jnp.int32, (_T, _T), 1)
    eye = (ri == ci).astype(f32)
    pn2_col = jnp.sum(pp * eye, axis=1, keepdims=True)
    pn2_row = jnp.sum(pp * eye, axis=0, keepdims=True)
    pn_col = jnp.maximum(jnp.sqrt(pn2_col), 1e-8)
    pn_row = jnp.maximum(jnp.sqrt(pn2_row), 1e-8)
    sim = pp / (pn_col * pn_row)

    a3 = sim[:, :, None]
    b3 = sim[:, None, :]
    ki = lax.broadcasted_iota(jnp.int32, (_T, _T, _T), 1)
    ji = lax.broadcasted_iota(jnp.int32, (_T, _T, _T), 2)
    gt = (a3 > b3).astype(f32)
    eq = jnp.logical_and(a3 == b3, ki < ji).astype(f32)
    rank = jnp.sum(gt + eq, axis=1)
    disc_ref[...] = jnp.log(rank + 2.0) * 1.4426950408889634


def _tc1_body(lab_ref, labsc_ref, x_ref, proto_ref, fsum_ref, cnt_ref,
              pp_ref, disc_ref):
    f32 = jnp.float32
    i = pl.program_id(0)

    @pl.when(i == 0)
    def _proto_side():
        _proto_rank(proto_ref, pp_ref, disc_ref)

    lab_row = lab_ref[0]
    lab3 = jnp.broadcast_to(lab_row, (_T, _TCB))
    ti = lax.broadcasted_iota(jnp.int32, (_T, _TCB), 0)
    oh_bool = lab3 == ti
    oht = oh_bool.astype(f32)

    part = lax.dot_general(oht, x_ref[...], (((1,), (0,)), ((), ())),
                           preferred_element_type=f32)
    pcnt = jnp.sum(oh_bool.astype(f32), axis=1, keepdims=True)

    @pl.when(i == 0)
    def _init():
        lsc = jnp.broadcast_to(labsc_ref[0], (_T, _N_SC))
        tisc = lax.broadcasted_iota(jnp.int32, (_T, _N_SC), 0)
        csc = jnp.sum((lsc == tisc).astype(f32), axis=1, keepdims=True)
        fsum_ref[...] = part
        cnt_ref[...] = pcnt + csc

    @pl.when(i > 0)
    def _acc():
        fsum_ref[...] = fsum_ref[...] + part
        cnt_ref[...] = cnt_ref[...] + pcnt


def _tc2_body(psum_ref, fsum_ref, cntc_ref, w_ref, b_ref,
              proto_ref, pp_ref, disc_ref, temp_ref, out_ref):
    f32 = jnp.float32
    featsum = jnp.sum(psum_ref[...], axis=0) + fsum_ref[...]
    counts = cntc_ref[...]
    sums = lax.dot_general(featsum, w_ref[...], (((1,), (1,)), ((), ())),
                           preferred_element_type=f32)
    sums = sums + counts * b_ref[...]
    means = sums / jnp.maximum(counts, 1.0)

    mp = lax.dot_general(means, proto_ref[...], (((1,), (1,)), ((), ())),
                         preferred_element_type=f32)
    pp = pp_ref[...]

    ri = lax.broadcasted_iota(jnp.int32, (_T, _T), 0)
    ci = lax.broadcasted_iota(jnp.int32, (_T, _T), 1)
    eye = (ri == ci).astype(f32)

    m2 = jnp.sum(means * means, axis=1, keepdims=True)
    nm = jnp.maximum(jnp.sqrt(m2), 1e-8)
    pn2_row = jnp.sum(pp * eye, axis=0, keepdims=True)
    pn_row = jnp.maximum(jnp.sqrt(pn2_row), 1e-8)

    cos_mp = mp / (nm * pn_row)
    temp = temp_ref[0, 0]
    apd = (-(1.0 - cos_mp) / temp) / disc_ref[...]

    rmax = jnp.max(apd, axis=1, keepdims=True)
    lse = jnp.log(jnp.sum(jnp.exp(apd - rmax), axis=1, keepdims=True)) + rmax
    diag_ap = jnp.sum(apd * eye, axis=1, keepdims=True)
    loss_i = lse - diag_ap
    present = (counts > 0).astype(f32)
    total = jnp.sum(loss_i * present, axis=0, keepdims=True)
    out_ref[...] = total / _T


def kernel(features, labels, W, b, proto, temperature=0.3):
    feat2d = features.reshape(_N, _D)
    lab2d = labels.astype(jnp.int32)

    psum = _sc_segsum(feat2d, lab2d)

    fsum_tc, cnt_tc, pp, disc = pl.pallas_call(
        _tc1_body,
        grid=(_TCG,),
        out_shape=(
            jax.ShapeDtypeStruct((_T, _D), jnp.float32),
            jax.ShapeDtypeStruct((_T, 1), jnp.float32),
            jax.ShapeDtypeStruct((_T, _T), jnp.float32),
            jax.ShapeDtypeStruct((_T, _T), jnp.float32),
        ),
        in_specs=[
            pl.BlockSpec((1, 1, _S), lambda i: (i + _N_SC // _S, 0, 0)),
            pl.BlockSpec((1, 1, _N_SC), lambda i: (0, 0, 0)),
            pl.BlockSpec((_TCB, _D), lambda i: (i + _N_SC // _TCB, 0)),
            pl.BlockSpec((_T, _T), lambda i: (0, 0)),
        ],
        out_specs=(
            pl.BlockSpec((_T, _D), lambda i: (0, 0)),
            pl.BlockSpec((_T, 1), lambda i: (0, 0)),
            pl.BlockSpec((_T, _T), lambda i: (0, 0)),
            pl.BlockSpec((_T, _T), lambda i: (0, 0)),
        ),
    )(lab2d.reshape(_B, 1, _S), lab2d.reshape(_B, 1, _S), feat2d, proto)

    b_row = b.reshape(1, _T).astype(jnp.float32)
    t11 = jnp.asarray(temperature, jnp.float32).reshape(1, 1)
    out = pl.pallas_call(
        _tc2_body,
        out_shape=jax.ShapeDtypeStruct((1, 1), jnp.float32),
        in_specs=[pl.BlockSpec(memory_space=pltpu.VMEM)] * 8
        + [pl.BlockSpec(memory_space=pltpu.SMEM)],
        out_specs=pl.BlockSpec(memory_space=pltpu.VMEM),
    )(psum, fsum_tc, cnt_tc, W.astype(jnp.float32), b_row,
      proto, pp, disc, t11)
    return out.reshape(1)
